# baseline (speedup 1.0000x reference)
import functools

import jax
import jax.numpy as jnp
import numpy as np
from jax import lax
from jax.experimental import pallas as pl
from jax.experimental.pallas import tpu as pltpu
from jax.experimental.pallas import tpu_sc as plsc

_C = 1000
_D = 10
_R = 8
_OCOLS = 1000
_PAD = [max(16, 1 << d) for d in range(_D + 1)]
_MOFF = {}
_o = 0
for _d in range(1, _D + 1):
    _MOFF[_d] = _o
    _o += _PAD[_d]
_MSZ = _o
_POFF = {}
_o = 0
for _d in range(1, _D):
    _POFF[_d] = _o
    _o += _PAD[_d]
_PSZ = _o


def _bitrev(i, d):
    r = 0
    for _ in range(d):
        r = (r << 1) | (i & 1)
        i >>= 1
    return r


def _build_tables():
    gphys = np.zeros(_PAD[_D], np.int32)
    scphys = np.zeros(_PAD[_D], np.int32)
    scmask = np.zeros(_PAD[_D], np.int32)
    cnt = {d: np.zeros(1 << d, np.float64) for d in range(1, _D + 1)}
    pa_h = {d: np.ones(1 << d, np.float32) for d in range(1, _D + 1)}
    pb_h = {d: np.zeros(1 << d, np.float32) for d in range(1, _D + 1)}

    def rec(a, b, d, p):
        cnt[d][p] = b - a
        if b - a == 1:
            q = p
            for dd in range(d + 1, _D + 1):
                q = 2 * q
                cnt[dd][q] = 1
                pa_h[dd][q] = 0.0
                pb_h[dd][q] = 1.0
            s = _bitrev(q, _D)
            gphys[s] = a
            scphys[s] = a
            scmask[s] = 1
        else:
            mid = a + (b - a) // 2
            rec(a, mid, d + 1, 2 * p)
            rec(mid, b, d + 1, 2 * p + 1)

    rec(0, _C // 2, 1, 0)
    rec(_C // 2, _C, 1, 1)

    we = {}
    wo = {}
    pa = {}
    pb = {}
    for d in range(1, _D + 1):
        n = 1 << d
        hid = np.array([_bitrev(p, d) for p in range(n)])
        pa[d] = np.ones(_PAD[d], np.float32)
        pb[d] = np.zeros(_PAD[d], np.float32)
        pa[d][:n] = pa_h[d][hid]
        pb[d][:n] = pb_h[d][hid]
        if d < _D:
            cl = cnt[d + 1][2 * hid]
            cr = cnt[d + 1][2 * hid + 1]
            cp = np.maximum(cnt[d][hid], 1)
            we[d] = np.zeros(_PAD[d], np.float32)
            wo[d] = np.zeros(_PAD[d], np.float32)
            we[d][:n] = np.where(cnt[d][hid] > 0, cl / cp, 0.0)
            wo[d][:n] = np.where(cnt[d][hid] > 0, cr / cp, 0.0)

    ioffs = {}
    iparts = []

    def iadd(name, arr):
        ioffs[name] = sum(len(x) for x in iparts)
        iparts.append(np.asarray(arr, np.int32))

    foffs = {}
    fparts = []

    def fadd(name, arr):
        foffs[name] = sum(len(x) for x in fparts)
        fparts.append(np.asarray(arr, np.float32))

    iadd("gphys", gphys)
    iadd("scphys", scphys)
    iadd("scmask", scmask)
    for d in range(1, _D):
        fadd(("we", d), we[d])
        fadd(("wo", d), wo[d])
    for d in range(1, _D + 1):
        fadd(("pa", d), pa[d])
        fadd(("pb", d), pb[d])

    pa_need = {
        d: [bool(np.any(pa[d][16 * k:16 * k + 16] != 1.0)) for k in range(_PAD[d] // 16)]
        for d in range(1, _D + 1)
    }
    mask_need = [bool(np.any(scmask[16 * k:16 * k + 16] == 0))
                 for k in range(_PAD[_D] // 16)]
    return (np.concatenate(iparts), np.concatenate(fparts), ioffs, foffs,
            pa_need, mask_need)


_ITAB, _FTAB, _IOFFS, _FOFFS, _PA_NEED, _MASK_NEED = _build_tables()

_INFO = plsc.get_sparse_core_info()
_NW = _INFO.num_cores * _INFO.num_subcores

_TAKE_DN = lax.GatherDimensionNumbers(
    offset_dims=(), collapsed_slice_dims=(0,), start_index_map=(0,))


def _take(v, idx):
    return lax.gather(v, idx[:, None], _TAKE_DN, slice_sizes=(1,),
                      mode=lax.GatherScatterMode.PROMISE_IN_BOUNDS)


def _groups(items, g):
    return [items[i:i + g] for i in range(0, len(items), g)]


def _block_program(rvecs, rbuf, obuf, mean_v, itv, ftv, lane):
    rows = range(_R)

    def li(name, k):
        return itv[pl.ds(_IOFFS[name] + 16 * k, 16)]

    def lf(name, k):
        return ftv[pl.ds(_FOFFS[name] + 16 * k, 16)]

    def mref(r, d, k):
        return mean_v[pl.ds(r * _MSZ + _MOFF[d] + 16 * k, 16)]

    h10 = _PAD[_D] // 32
    for k in range(h10):
        gi = {kk: li("gphys", kk) for kk in (k, k + h10)}
        we9 = lf(("we", _D - 1), k)
        wo9 = lf(("wo", _D - 1), k)
        va = {r: plsc.load_gather(rbuf, [rvecs[r], gi[k]]) for r in rows}
        vb = {r: plsc.load_gather(rbuf, [rvecs[r], gi[k + h10]]) for r in rows}
        m9 = {r: va[r] * we9 + vb[r] * wo9 for r in rows}
        for r in rows:
            mean_v[pl.ds(r * _MSZ + _MOFF[_D] + 16 * k, 16)] = va[r]
            mean_v[pl.ds(r * _MSZ + _MOFF[_D] + 16 * (k + h10), 16)] = vb[r]
            mean_v[pl.ds(r * _MSZ + _MOFF[_D - 1] + 16 * k, 16)] = m9[r]

    for d in range(_D - 2, 0, -1):
        nch = _PAD[d] // 16
        for ks in _groups(list(range(nch)), 2):
            we = {k: lf(("we", d), k) for k in ks}
            wo = {k: lf(("wo", d), k) for k in ks}
            units = [(k, r) for k in ks for r in rows]
            res = {}
            if (1 << (d + 1)) <= 16:
                odp = (lane + (1 << d)) & 15
                for k, r in units:
                    c = mref(r, d + 1, 0)
                    res[(k, r)] = c * we[k] + _take(c, odp) * wo[k]
            else:
                half1 = nch
                for k, r in units:
                    cl = mref(r, d + 1, k)
                    cr = mref(r, d + 1, k + half1)
                    res[(k, r)] = cl * we[k] + cr * wo[k]
            for k, r in units:
                mean_v[pl.ds(r * _MSZ + _MOFF[d] + 16 * k, 16)] = res[(k, r)]

    def sig_pair(d, k, rset):
        h = _PAD[d] // 32
        pav = {}
        pbv = {}
        for kk in (k, k + h):
            if _PA_NEED[d][kk]:
                pav[kk] = lf(("pa", d), kk)
                pbv[kk] = lf(("pb", d), kk)
        sA = {r: mref(r, d, k) for r in rset}
        sB = {r: mref(r, d, k + h) for r in rset}
        e = {r: jnp.exp(sB[r] - sA[r]) for r in rset}
        pA = {r: 1.0 / (1.0 + e[r]) for r in rset}
        pB = {r: 1.0 - pA[r] for r in rset}
        if k in pav:
            pA = {r: pA[r] * pav[k] + pbv[k] for r in rset}
        if k + h in pav:
            pB = {r: pB[r] * pav[k + h] + pbv[k + h] for r in rset}
        return pA, pB

    def dfs(d, k, par, rset):
        h = _PAD[d] // 32
        pA, pB = sig_pair(d, k, rset)
        ppA = {r: pA[r] * par[r] for r in rset}
        ppB = {r: pB[r] * par[r] for r in rset}
        if d == _D:
            sidx = {}
            smask = {}
            for kk in (k, k + h):
                sidx[kk] = li("scphys", kk)
                smask[kk] = li("scmask", kk) != 0 if _MASK_NEED[kk] else None
            for r in rset:
                plsc.store_scatter(obuf, [rvecs[r], sidx[k]], ppA[r],
                                   mask=smask[k])
                plsc.store_scatter(obuf, [rvecs[r], sidx[k + h]], ppB[r],
                                   mask=smask[k + h])
        else:
            dfs(d + 1, k, ppA, rset)
            dfs(d + 1, k + h, ppB, rset)

    for rset in _groups(list(rows), 8):
        pp = None
        for d in range(1, 5):
            sibp = lane ^ (1 << (d - 1))
            parp = lane & ((1 << (d - 1)) - 1)
            pa_c = lf(("pa", d), 0) if _PA_NEED[d][0] else None
            nxt = {}
            for r in rset:
                s = mref(r, d, 0)
                sv = _take(s, sibp)
                p = 1.0 / (1.0 + jnp.exp(sv - s))
                if pa_c is not None:
                    p = p * pa_c + lf(("pb", d), 0)
                if d > 1:
                    p = p * _take(pp[r], parp)
                nxt[r] = p
            pp = nxt
        dfs(5, 0, pp, rset)


def _sc_body(rows_per_w, x_hbm, itab_hbm, ftab_hbm, out_hbm,
             itv, ftv, rbuf, obuf, mean_v, sin0, sin1, sout0, sout1):
    wid = lax.axis_index("s") * _INFO.num_cores + lax.axis_index("c")
    pltpu.sync_copy(itab_hbm, itv)
    pltpu.sync_copy(ftab_hbm, ftv)
    row0 = wid * rows_per_w
    nblk = rows_per_w // _R

    lane = lax.iota(jnp.int32, 16)

    def in_copy(blk, par, sem):
        pltpu.async_copy(x_hbm.at[pl.ds(row0 + blk * _R, _R)],
                         rbuf.at[pl.ds(par * _R, _R)], sem)

    def in_wait(sem):
        pltpu.make_async_copy(x_hbm.at[pl.ds(0, _R)],
                              rbuf.at[pl.ds(0, _R)], sem).wait()

    def out_copy(blk, par, sem):
        pltpu.async_copy(obuf.at[pl.ds(par * _R, _R)],
                         out_hbm.at[pl.ds(row0 + blk * _R, _R)], sem)

    def out_wait(sem):
        pltpu.make_async_copy(obuf.at[pl.ds(0, _R)],
                              out_hbm.at[pl.ds(0, _R)], sem).wait()

    in_copy(0, 0, sin0)
    in_copy(1, 1, sin1)

    def body(j, carry):
        par = j & 1
        rvecs = [jnp.full((16,), 0, jnp.int32) + (par * _R + r) for r in range(_R)]

        @pl.when(par == 0)
        def _():
            in_wait(sin0)

        @pl.when(par == 1)
        def _():
            in_wait(sin1)

        _block_program(rvecs, rbuf, obuf, mean_v, itv, ftv, lane)

        @pl.when(jnp.logical_and(j + 2 < nblk, par == 0))
        def _():
            in_copy(j + 2, 0, sin0)

        @pl.when(jnp.logical_and(j + 2 < nblk, par == 1))
        def _():
            in_copy(j + 2, 1, sin1)

        @pl.when(jnp.logical_and(j >= 2, par == 0))
        def _():
            out_wait(sout0)

        @pl.when(jnp.logical_and(j >= 2, par == 1))
        def _():
            out_wait(sout1)

        @pl.when(par == 0)
        def _():
            out_copy(j, 0, sout0)

        @pl.when(par == 1)
        def _():
            out_copy(j, 1, sout1)

        return carry

    lax.fori_loop(0, nblk, body, 0)
    out_wait(sout0)
    out_wait(sout1)


@jax.jit
def kernel(outputs):
    B = outputs.shape[0]
    assert B % (_R * _NW) == 0
    rows_per_w = B // _NW
    mesh = plsc.VectorSubcoreMesh(core_axis_name="c", subcore_axis_name="s")
    fn = pl.kernel(
        functools.partial(_sc_body, rows_per_w),
        out_type=jax.ShapeDtypeStruct((B, _C), jnp.float32),
        mesh=mesh,
        scratch_types=[
            pltpu.VMEM((len(_ITAB),), jnp.int32),
            pltpu.VMEM((len(_FTAB),), jnp.float32),
            pltpu.VMEM((2 * _R, _C), jnp.float32),
            pltpu.VMEM((2 * _R, _OCOLS), jnp.float32),
            pltpu.VMEM((_R * _MSZ,), jnp.float32),
            pltpu.SemaphoreType.DMA,
            pltpu.SemaphoreType.DMA,
            pltpu.SemaphoreType.DMA,
            pltpu.SemaphoreType.DMA,
        ],
        compiler_params=pltpu.CompilerParams(needs_layout_passes=False),
    )
    return fn(outputs, jnp.asarray(_ITAB), jnp.asarray(_FTAB))


# speedup vs baseline: 1.0628x; 1.0628x over previous
import functools

import jax
import jax.numpy as jnp
import numpy as np
from jax import lax
from jax.experimental import pallas as pl
from jax.experimental.pallas import tpu as pltpu
from jax.experimental.pallas import tpu_sc as plsc

_C = 1000
_D = 10
_R = 8
_OCOLS = 1000
_PAD = [max(16, 1 << d) for d in range(_D + 1)]
_MOFF = {}
_o = 0
for _d in range(1, _D + 1):
    _MOFF[_d] = _o
    _o += _PAD[_d]
_MSZ = _o
_POFF = {}
_o = 0
for _d in range(1, _D):
    _POFF[_d] = _o
    _o += _PAD[_d]
_PSZ = _o


def _bitrev(i, d):
    r = 0
    for _ in range(d):
        r = (r << 1) | (i & 1)
        i >>= 1
    return r


def _build_tables():
    gphys = np.zeros(_PAD[_D], np.int32)
    scphys = np.zeros(_PAD[_D], np.int32)
    scmask = np.zeros(_PAD[_D], np.int32)
    cnt = {d: np.zeros(1 << d, np.float64) for d in range(1, _D + 1)}
    pa_h = {d: np.ones(1 << d, np.float32) for d in range(1, _D + 1)}
    pb_h = {d: np.zeros(1 << d, np.float32) for d in range(1, _D + 1)}

    def rec(a, b, d, p):
        cnt[d][p] = b - a
        if b - a == 1:
            q = p
            for dd in range(d + 1, _D + 1):
                q = 2 * q
                cnt[dd][q] = 1
                pa_h[dd][q] = 0.0
                pb_h[dd][q] = 1.0
            s = _bitrev(q, _D)
            gphys[s] = a
            scphys[s] = a
            scmask[s] = 1
        else:
            mid = a + (b - a) // 2
            rec(a, mid, d + 1, 2 * p)
            rec(mid, b, d + 1, 2 * p + 1)

    rec(0, _C // 2, 1, 0)
    rec(_C // 2, _C, 1, 1)

    we = {}
    wo = {}
    pa = {}
    pb = {}
    for d in range(1, _D + 1):
        n = 1 << d
        hid = np.array([_bitrev(p, d) for p in range(n)])
        pa[d] = np.ones(_PAD[d], np.float32)
        pb[d] = np.zeros(_PAD[d], np.float32)
        pa[d][:n] = pa_h[d][hid]
        pb[d][:n] = pb_h[d][hid]
        if d < _D:
            cl = cnt[d + 1][2 * hid]
            cr = cnt[d + 1][2 * hid + 1]
            cp = np.maximum(cnt[d][hid], 1)
            we[d] = np.zeros(_PAD[d], np.float32)
            wo[d] = np.zeros(_PAD[d], np.float32)
            we[d][:n] = np.where(cnt[d][hid] > 0, cl / cp, 0.0)
            wo[d][:n] = np.where(cnt[d][hid] > 0, cr / cp, 0.0)

    ioffs = {}
    iparts = []

    def iadd(name, arr):
        ioffs[name] = sum(len(x) for x in iparts)
        iparts.append(np.asarray(arr, np.int32))

    foffs = {}
    fparts = []

    def fadd(name, arr):
        foffs[name] = sum(len(x) for x in fparts)
        fparts.append(np.asarray(arr, np.float32))

    iadd("gphys", gphys)
    iadd("scphys", scphys)
    iadd("scmask", scmask)
    for d in range(1, _D):
        fadd(("we", d), we[d])
        fadd(("wo", d), wo[d])
    for d in range(1, _D + 1):
        fadd(("pa", d), pa[d])
        fadd(("pb", d), pb[d])

    pa_need = {
        d: [bool(np.any(pa[d][16 * k:16 * k + 16] != 1.0)) for k in range(_PAD[d] // 16)]
        for d in range(1, _D + 1)
    }
    mask_need = [bool(np.any(scmask[16 * k:16 * k + 16] == 0))
                 for k in range(_PAD[_D] // 16)]
    return (np.concatenate(iparts), np.concatenate(fparts), ioffs, foffs,
            pa_need, mask_need)


_ITAB, _FTAB, _IOFFS, _FOFFS, _PA_NEED, _MASK_NEED = _build_tables()

_INFO = plsc.get_sparse_core_info()
_NW = _INFO.num_cores * _INFO.num_subcores

_TAKE_DN = lax.GatherDimensionNumbers(
    offset_dims=(), collapsed_slice_dims=(0,), start_index_map=(0,))


def _take(v, idx):
    return lax.gather(v, idx[:, None], _TAKE_DN, slice_sizes=(1,),
                      mode=lax.GatherScatterMode.PROMISE_IN_BOUNDS)


def _groups(items, g):
    return [items[i:i + g] for i in range(0, len(items), g)]


def _block_program(rvecs, rbuf, obuf, mean_v, pp_v, itv, ftv, lane):
    rows = range(_R)

    def li(name, k):
        return itv[pl.ds(_IOFFS[name] + 16 * k, 16)]

    def lf(name, k):
        return ftv[pl.ds(_FOFFS[name] + 16 * k, 16)]

    def mref(r, d, k):
        return mean_v[pl.ds(r * _MSZ + _MOFF[d] + 16 * k, 16)]

    def pref(r, d, k):
        return pp_v[pl.ds(r * _PSZ + _POFF[d] + 16 * k, 16)]

    h10 = _PAD[_D] // 32
    for k in range(h10):
        gi = {kk: li("gphys", kk) for kk in (k, k + h10)}
        we9 = lf(("we", _D - 1), k)
        wo9 = lf(("wo", _D - 1), k)
        va = {r: plsc.load_gather(rbuf, [rvecs[r], gi[k]]) for r in rows}
        vb = {r: plsc.load_gather(rbuf, [rvecs[r], gi[k + h10]]) for r in rows}
        m9 = {r: va[r] * we9 + vb[r] * wo9 for r in rows}
        for r in rows:
            mean_v[pl.ds(r * _MSZ + _MOFF[_D] + 16 * k, 16)] = va[r]
            mean_v[pl.ds(r * _MSZ + _MOFF[_D] + 16 * (k + h10), 16)] = vb[r]
            mean_v[pl.ds(r * _MSZ + _MOFF[_D - 1] + 16 * k, 16)] = m9[r]

    for d in range(_D - 2, 0, -1):
        nch = _PAD[d] // 16
        for ks in _groups(list(range(nch)), 2):
            we = {k: lf(("we", d), k) for k in ks}
            wo = {k: lf(("wo", d), k) for k in ks}
            units = [(k, r) for k in ks for r in rows]
            res = {}
            if (1 << (d + 1)) <= 16:
                odp = (lane + (1 << d)) & 15
                for k, r in units:
                    c = mref(r, d + 1, 0)
                    res[(k, r)] = c * we[k] + _take(c, odp) * wo[k]
            else:
                half1 = nch
                for k, r in units:
                    cl = mref(r, d + 1, k)
                    cr = mref(r, d + 1, k + half1)
                    res[(k, r)] = cl * we[k] + cr * wo[k]
            for k, r in units:
                mean_v[pl.ds(r * _MSZ + _MOFF[d] + 16 * k, 16)] = res[(k, r)]

    for d in range(1, _D + 1):
        nch = _PAD[d] // 16
        if (1 << d) <= 16:
            sibp = lane ^ (1 << (d - 1))
            parp = lane & ((1 << (d - 1)) - 1)
            for r in rows:
                s = mref(r, d, 0)
                sv = _take(s, sibp)
                p = 1.0 / (1.0 + jnp.exp(sv - s))
                if _PA_NEED[d][0]:
                    p = p * lf(("pa", d), 0) + lf(("pb", d), 0)
                if d > 1:
                    p = p * _take(pref(r, d - 1, 0), parp)
                pp_v[pl.ds(r * _PSZ + _POFF[d], 16)] = p
        else:
            h = nch // 2
            for ks in _groups(list(range(h)), 2):
                units = [(k, r) for k in ks for r in rows]
                pav = {}
                pbv = {}
                for k in ks:
                    for kk in (k, k + h):
                        if _PA_NEED[d][kk]:
                            pav[kk] = lf(("pa", d), kk)
                            pbv[kk] = lf(("pb", d), kk)
                sidx = {}
                smask = {}
                if d == _D:
                    for k in ks:
                        for kk in (k, k + h):
                            sidx[kk] = li("scphys", kk)
                            if _MASK_NEED[kk]:
                                smask[kk] = li("scmask", kk) != 0
                sA = {u: mref(u[1], d, u[0]) for u in units}
                sB = {u: mref(u[1], d, u[0] + h) for u in units}
                e = {u: jnp.exp(sB[u] - sA[u]) for u in units}
                pA = {u: 1.0 / (1.0 + e[u]) for u in units}
                pB = {u: 1.0 - pA[u] for u in units}
                for k, r in units:
                    u = (k, r)
                    if k in pav:
                        pA[u] = pA[u] * pav[k] + pbv[k]
                    if k + h in pav:
                        pB[u] = pB[u] * pav[k + h] + pbv[k + h]
                if d > 1:
                    par = {u: pref(u[1], d - 1, u[0]) for u in units}
                    pA = {u: pA[u] * par[u] for u in units}
                    pB = {u: pB[u] * par[u] for u in units}
                for k, r in units:
                    u = (k, r)
                    if d < _D:
                        pp_v[pl.ds(r * _PSZ + _POFF[d] + 16 * k, 16)] = pA[u]
                        pp_v[pl.ds(r * _PSZ + _POFF[d] + 16 * (k + h), 16)] = pB[u]
                    else:
                        plsc.store_scatter(obuf, [rvecs[r], sidx[k]], pA[u],
                                           mask=smask.get(k))
                        plsc.store_scatter(obuf, [rvecs[r], sidx[k + h]], pB[u],
                                           mask=smask.get(k + h))


def _sc_body(rows_per_w, x_hbm, itab_hbm, ftab_hbm, out_hbm,
             itv, ftv, rbuf, obuf, mean_v, pp_v, sin0, sin1, sout0, sout1):
    wid = lax.axis_index("s") * _INFO.num_cores + lax.axis_index("c")
    pltpu.sync_copy(itab_hbm, itv)
    pltpu.sync_copy(ftab_hbm, ftv)
    row0 = wid * rows_per_w
    nblk = rows_per_w // _R

    lane = lax.iota(jnp.int32, 16)

    def in_copy(blk, par, sem):
        pltpu.async_copy(x_hbm.at[pl.ds(row0 + blk * _R, _R)],
                         rbuf.at[pl.ds(par * _R, _R)], sem)

    def in_wait(sem):
        pltpu.make_async_copy(x_hbm.at[pl.ds(0, _R)],
                              rbuf.at[pl.ds(0, _R)], sem).wait()

    def out_copy(blk, par, sem):
        pltpu.async_copy(obuf.at[pl.ds(par * _R, _R)],
                         out_hbm.at[pl.ds(row0 + blk * _R, _R)], sem)

    def out_wait(sem):
        pltpu.make_async_copy(obuf.at[pl.ds(0, _R)],
                              out_hbm.at[pl.ds(0, _R)], sem).wait()

    in_copy(0, 0, sin0)
    in_copy(1, 1, sin1)

    def body(j, carry):
        par = j & 1
        rvecs = [jnp.full((16,), 0, jnp.int32) + (par * _R + r) for r in range(_R)]

        @pl.when(par == 0)
        def _():
            in_wait(sin0)

        @pl.when(par == 1)
        def _():
            in_wait(sin1)

        _block_program(rvecs, rbuf, obuf, mean_v, pp_v, itv, ftv, lane)

        @pl.when(jnp.logical_and(j + 2 < nblk, par == 0))
        def _():
            in_copy(j + 2, 0, sin0)

        @pl.when(jnp.logical_and(j + 2 < nblk, par == 1))
        def _():
            in_copy(j + 2, 1, sin1)

        @pl.when(jnp.logical_and(j >= 2, par == 0))
        def _():
            out_wait(sout0)

        @pl.when(jnp.logical_and(j >= 2, par == 1))
        def _():
            out_wait(sout1)

        @pl.when(par == 0)
        def _():
            out_copy(j, 0, sout0)

        @pl.when(par == 1)
        def _():
            out_copy(j, 1, sout1)

        return carry

    lax.fori_loop(0, nblk, body, 0)
    out_wait(sout0)
    out_wait(sout1)


@jax.jit
def kernel(outputs):
    B = outputs.shape[0]
    assert B % (_R * _NW) == 0
    rows_per_w = B // _NW
    mesh = plsc.VectorSubcoreMesh(core_axis_name="c", subcore_axis_name="s")
    fn = pl.kernel(
        functools.partial(_sc_body, rows_per_w),
        out_type=jax.ShapeDtypeStruct((B, _C), jnp.float32),
        mesh=mesh,
        scratch_types=[
            pltpu.VMEM((len(_ITAB),), jnp.int32),
            pltpu.VMEM((len(_FTAB),), jnp.float32),
            pltpu.VMEM((2 * _R, _C), jnp.float32),
            pltpu.VMEM((2 * _R, _OCOLS), jnp.float32),
            pltpu.VMEM((_R * _MSZ,), jnp.float32),
            pltpu.VMEM((_R * _PSZ,), jnp.float32),
            pltpu.SemaphoreType.DMA,
            pltpu.SemaphoreType.DMA,
            pltpu.SemaphoreType.DMA,
            pltpu.SemaphoreType.DMA,
        ],
        compiler_params=pltpu.CompilerParams(needs_layout_passes=False),
    )
    return fn(outputs, jnp.asarray(_ITAB), jnp.asarray(_FTAB))
